# Initial kernel scaffold; baseline (speedup 1.0000x reference)
import jax
import jax.numpy as jnp
from jax.experimental import pallas as pl


def kernel(prop_state, annotation, A, W_gcn, b_gcn, W_out, b_out):
    raise NotImplementedError("write your pallas kernel here")


# speedup vs baseline: 2.6264x; 2.6264x over previous
import functools

import jax
import jax.numpy as jnp
from jax import lax
from jax.experimental import pallas as pl
from jax.experimental.pallas import tpu as pltpu
from jax.experimental.pallas import tpu_sc as plsc

N = 1024
NSLOT = 3
SD = 256
B = 4
NNZ = 32768
TLO = (NSLOT - 1) * N

NCORE = 2
NSUB = 16
EPT = NNZ // NSUB
VPB = EPT // 16
ZSL = (N * N) // NSUB
ZCH = 4096
BPC = B // NCORE


def _sc_body(rows_hbm, cols_hbm, vals_hbm, adj_hbm, deg_hbm,
             zeros_v, rows_v, cols_v, vals_v, idx_v, idxd_v, sval_v,
             adj_sh, deg_sh):
    c = lax.axis_index("c")
    s = lax.axis_index("s")

    def zinit(i, carry):
        zeros_v[pl.ds(i * 16, 16)] = jnp.zeros((16,), jnp.float32)
        return carry
    lax.fori_loop(0, ZCH // 16, zinit, 0)

    for bi in range(BPC):
        b = c * BPC + bi
        def zslice(k, carry):
            pltpu.sync_copy(zeros_v, adj_sh.at[pl.ds(s * ZSL + k * ZCH, ZCH)])
            return carry
        lax.fori_loop(0, ZSL // ZCH, zslice, 0)

        @pl.when(s == 0)
        def _zero_deg():
            pltpu.sync_copy(zeros_v.at[pl.ds(0, N)], deg_sh)

        e0 = s * EPT
        pltpu.sync_copy(rows_hbm.at[b, pl.ds(e0, EPT)], rows_v)
        pltpu.sync_copy(cols_hbm.at[b, pl.ds(e0, EPT)], cols_v)
        pltpu.sync_copy(vals_hbm.at[b, pl.ds(e0, EPT)], vals_v)

        def step(i, carry):
            kb = i * 16
            r = rows_v[pl.ds(kb, 16)]
            cc = cols_v[pl.ds(kb, 16)]
            v = vals_v[pl.ds(kb, 16)]
            m = (cc >= TLO) & (cc < TLO + N)
            flat = r * N + (cc - TLO)
            zi = jnp.zeros((16,), jnp.int32)
            idx_v[pl.ds(kb, 16)] = jnp.where(m, flat, zi)
            idxd_v[pl.ds(kb, 16)] = jnp.where(m, r, zi)
            sval_v[pl.ds(kb, 16)] = jnp.where(m, v, jnp.zeros((16,), jnp.float32))
            return carry
        lax.fori_loop(0, VPB, step, 0)

        plsc.subcore_barrier()

        pltpu.sync_copy(sval_v, adj_sh.at[idx_v], add=True)
        pltpu.sync_copy(sval_v, deg_sh.at[idxd_v], add=True)
        plsc.subcore_barrier()

        pltpu.sync_copy(adj_sh.at[pl.ds(s * ZSL, ZSL)],
                        adj_hbm.at[b, pl.ds(s * ZSL, ZSL)])

        @pl.when(s == 0)
        def _out_deg():
            pltpu.sync_copy(deg_sh, deg_hbm.at[b])


_sc_scatter = functools.partial(
    pl.kernel,
    out_type=(jax.ShapeDtypeStruct((B, N * N), jnp.float32),
              jax.ShapeDtypeStruct((B, N), jnp.float32)),
    mesh=plsc.VectorSubcoreMesh(core_axis_name="c", subcore_axis_name="s",
                                num_cores=NCORE, num_subcores=NSUB),
    scratch_types=[
        pltpu.VMEM((ZCH,), jnp.float32),
        pltpu.VMEM((EPT,), jnp.int32),
        pltpu.VMEM((EPT,), jnp.int32),
        pltpu.VMEM((EPT,), jnp.float32),
        pltpu.VMEM((EPT,), jnp.int32),
        pltpu.VMEM((EPT,), jnp.int32),
        pltpu.VMEM((EPT,), jnp.float32),
        pltpu.VMEM_SHARED((N * N,), jnp.float32),
        pltpu.VMEM_SHARED((N,), jnp.float32),
    ],
)(_sc_body)


R1 = 128


def _tc1_body(deg_ref, degb_ref, adj_ref, h_ref, hb_ref, wg_ref, bg_ref,
              on_ref):
    dinv = lax.rsqrt(deg_ref[0, 0] + 1.0)
    dinv_b = lax.rsqrt(degb_ref[0, 0] + 1.0)
    hn = h_ref[0] * dinv[:, None]
    ah = lax.dot_general(adj_ref[0], hn, (((1,), (0,)), ((), ())),
                         preferred_element_type=jnp.float32)
    ah = (ah + hb_ref[0] * dinv_b[:, None]) * dinv_b[:, None]
    z = lax.dot_general(ah, wg_ref[...], (((1,), (1,)), ((), ())),
                        preferred_element_type=jnp.float32)
    z = z + bg_ref[0][None, :]
    out = jnp.maximum(z, 0.0)
    nrm = jnp.sqrt(jnp.sum(out * out, axis=1))
    on_ref[0] = out / jnp.maximum(nrm, 1e-8)[:, None]


_tc1 = pl.pallas_call(
    _tc1_body,
    grid=(B, N // R1),
    in_specs=[
        pl.BlockSpec((1, 1, N), lambda b, rb: (b, 0, 0)),
        pl.BlockSpec((1, 1, R1), lambda b, rb: (b, 0, rb)),
        pl.BlockSpec((1, R1, N), lambda b, rb: (b, rb, 0)),
        pl.BlockSpec((1, N, SD), lambda b, rb: (b, 0, 0)),
        pl.BlockSpec((1, R1, SD), lambda b, rb: (b, rb, 0)),
        pl.BlockSpec((SD, SD), lambda b, rb: (0, 0)),
        pl.BlockSpec((1, SD), lambda b, rb: (0, 0)),
    ],
    out_specs=pl.BlockSpec((1, R1, SD), lambda b, rb: (b, rb, 0)),
    out_shape=jax.ShapeDtypeStruct((B, N, SD), jnp.float32),
)


R2 = 128


def _tc2_body(onb_ref, on_ref, wo_ref, bo_ref, out_ref):
    sim = lax.dot_general(onb_ref[0], on_ref[0], (((1,), (1,)), ((), ())),
                          preferred_element_type=jnp.float32)
    logits = lax.dot_general(sim, wo_ref[...], (((1,), (1,)), ((), ())),
                             preferred_element_type=jnp.float32)
    logits = logits + bo_ref[0][None, :]
    out_ref[0] = 1.0 / (1.0 + jnp.exp(-logits))


_tc2 = pl.pallas_call(
    _tc2_body,
    grid=(B, N // R2),
    in_specs=[
        pl.BlockSpec((1, R2, SD), lambda b, rb: (b, rb, 0)),
        pl.BlockSpec((1, N, SD), lambda b, rb: (b, 0, 0)),
        pl.BlockSpec((N, N), lambda b, rb: (0, 0)),
        pl.BlockSpec((1, N), lambda b, rb: (0, 0)),
    ],
    out_specs=pl.BlockSpec((1, R2, N), lambda b, rb: (b, rb, 0)),
    out_shape=jax.ShapeDtypeStruct((B, N, N), jnp.float32),
)


def kernel(prop_state, annotation, A, W_gcn, b_gcn, W_out, b_out):
    del annotation
    rows = A[:, 0, 1].astype(jnp.int32)
    cols = A[:, 0, 2].astype(jnp.int32)
    vals = A[:, 0, 0]
    adj_flat, deg = _sc_scatter(rows, cols, vals)
    adj = adj_flat.reshape(B, N, N)
    deg3 = deg.reshape(B, 1, N)
    h = prop_state[:, :, NSLOT - 1]
    on = _tc1(deg3, deg3, adj, h, h, W_gcn, b_gcn.reshape(1, SD))
    return _tc2(on, on, W_out, b_out.reshape(1, N))
